# baseline (speedup 1.0000x reference)
import functools

import jax
import jax.numpy as jnp
from jax import lax
from jax.experimental import pallas as pl
from jax.experimental.pallas import tpu as pltpu
from jax.experimental.pallas import tpu_sc as plsc

N = 16384
C_CAT = 26
C_NUM = 13
V = 100000
D = 16
NC = 2
NS = 16
NW = NC * NS

DLEN = 50048
TAIL = 32
HLEN = DLEN + TAIL
HOFF1 = V - HLEN
BSPLIT = 50000
IDX_P = 8192
NUM_P = 2048


def _sc_body(embT_u, embT_i, idxT, numT, lwb, tails, outp,
             acc_v, idx_v, lw_v, tl_v, row0_v, row1_v, vsem0, vsem1, isem):
    wid = lax.axis_index("s") * NC + lax.axis_index("c")
    vh = wid // D
    d = wid % D
    lo = vh * BSPLIT
    hi = lo + BSPLIT
    off = vh * HOFF1
    zero16 = jnp.zeros((D,), jnp.int32)

    def phase(tt, embT):
        pltpu.sync_copy(lwb.at[tt * NW + wid], lw_v)
        wk = [lw_v[pl.ds(k * D, D)] for k in range(C_NUM + 1)]
        pltpu.sync_copy(tails.at[tt * D + d], tl_v)

        def init_scope(num_v):
            def ibody(p, _):
                pltpu.sync_copy(
                    numT.at[pl.ds(tt * D, D), pl.ds(p * NUM_P, NUM_P)],
                    num_v)

                def nbody(j, _):
                    val = num_v[0, pl.ds(j * D, D)] * wk[0]
                    for k in range(1, C_NUM + 1):
                        val = val + num_v[k, pl.ds(j * D, D)] * wk[k]
                    acc_v[pl.ds(p * NUM_P + j * D, D)] = val
                    return 0

                lax.fori_loop(0, NUM_P // D, nbody, 0, unroll=4)
                return 0

            lax.fori_loop(0, N // NUM_P, ibody, 0)

        pl.run_scoped(init_scope, pltpu.VMEM((D, NUM_P), jnp.float32))

        def main_scope(b0, b1):
            def vstart(c, buf, row_v, sem):
                row_v[pl.ds(0, D)] = jnp.full((D,), c * D + d, jnp.int32)
                pltpu.async_copy(
                    embT.at[:, pl.ds(off, DLEN)].at[row_v.at[pl.ds(0, 1)]],
                    buf.at[:, pl.ds(0, DLEN)], sem)

            def vwait(buf, sem):
                pltpu.make_async_copy(
                    embT.at[:, pl.ds(off, DLEN)].at[row0_v.at[pl.ds(0, 1)]],
                    buf.at[:, pl.ds(0, DLEN)], sem).wait()

            def col(c, buf):
                for q in range(TAIL // D):
                    buf[0, pl.ds(DLEN + q * D, D)] = tl_v[c, 0, pl.ds(q * D, D)]
                for h in range(N // IDX_P):
                    pltpu.sync_copy(idxT.at[tt, c, h], idx_v)
                    base = h * IDX_P

                    def gbody(j, _):
                        idxv = idx_v[pl.ds(j * D, D)]
                        m = (idxv >= lo) & (idxv < hi)
                        rel = jnp.where(m, idxv - off, 0)
                        g = plsc.load_gather(buf, [zero16, rel])
                        a = base + j * D
                        acc_v[pl.ds(a, D)] = (acc_v[pl.ds(a, D)]
                                              + jnp.where(m, g, 0.0))
                        return 0

                    lax.fori_loop(0, IDX_P // D, gbody, 0, unroll=2)

            vstart(0, b0, row0_v, vsem0)

            def cbody(cc, _):
                c0 = cc * 2
                vwait(b0, vsem0)
                vstart(c0 + 1, b1, row1_v, vsem1)
                col(c0, b0)
                vwait(b1, vsem1)
                vstart(jnp.minimum(c0 + 2, C_CAT - 1), b0, row0_v, vsem0)
                col(c0 + 1, b1)
                return 0

            lax.fori_loop(0, C_CAT // 2, cbody, 0)
            vwait(b0, vsem0)

        pl.run_scoped(main_scope, pltpu.VMEM((1, HLEN), jnp.float32),
                      pltpu.VMEM((1, HLEN), jnp.float32))

        pltpu.sync_copy(acc_v, outp.at[tt * NW + wid])

    phase(0, embT_u)
    phase(1, embT_i)


@jax.jit
def _run(embT_u, embT_i, idxT, numT, lwb, tails):
    mesh = plsc.VectorSubcoreMesh(core_axis_name="c", subcore_axis_name="s")
    return pl.kernel(
        _sc_body,
        out_type=jax.ShapeDtypeStruct((2 * NW, N), jnp.float32),
        mesh=mesh,
        scratch_types=[
            pltpu.VMEM((N,), jnp.float32),
            pltpu.VMEM((IDX_P,), jnp.int32),
            pltpu.VMEM((D * D,), jnp.float32),
            pltpu.VMEM((C_CAT, 1, TAIL), jnp.float32),
            pltpu.VMEM((D,), jnp.int32),
            pltpu.VMEM((D,), jnp.int32),
            pltpu.SemaphoreType.DMA,
            pltpu.SemaphoreType.DMA,
            pltpu.SemaphoreType.DMA,
        ],
        compiler_params=pltpu.CompilerParams(needs_layout_passes=False),
    )(embT_u, embT_i, idxT, numT, lwb, tails)


def kernel(cat_user, num_user, cat_item, num_item,
           emb_user, lin_w_user, lin_b_user,
           emb_item, lin_w_item, lin_b_item):
    embT_u = emb_user.swapaxes(1, 2).reshape(C_CAT * D, V)
    embT_i = emb_item.swapaxes(1, 2).reshape(C_CAT * D, V)
    idxT = jnp.stack([cat_user.astype(jnp.int32).T,
                      cat_item.astype(jnp.int32).T])
    idxT = idxT.reshape(2, C_CAT, N // IDX_P, IDX_P)
    ones = jnp.ones((1, N), jnp.float32)
    zer = jnp.zeros((D - C_NUM - 1, N), jnp.float32)
    numT = jnp.concatenate(
        [num_user.T, ones, zer, num_item.T, ones, zer], axis=0)
    zw = jnp.zeros((D - C_NUM - 1, D), jnp.float32)
    lw_u = jnp.concatenate([lin_w_user, lin_b_user.sum(0)[None], zw], axis=0)
    lw_i = jnp.concatenate([lin_w_item, lin_b_item.sum(0)[None], zw], axis=0)
    lw2 = jnp.stack([lw_u, lw_i])
    lwb4 = jnp.repeat(lw2.transpose(0, 2, 1)[:, :, :, None], D, axis=3)
    lwb4 = lwb4.reshape(2, 1, D, D * D)
    lwb = jnp.concatenate(
        [lwb4, jnp.zeros_like(lwb4)], axis=1).reshape(2 * NW, D * D)
    tails = jnp.stack([emb_user[:, V - TAIL:, :], emb_item[:, V - TAIL:, :]])
    tails = tails.transpose(0, 3, 1, 2).reshape(2 * D, C_CAT, 1, TAIL)
    outp = _run(embT_u, embT_i, idxT, numT, lwb, tails)
    p = outp.reshape(2, 2, D, N)
    return (p[:, 0] + p[:, 1]).swapaxes(1, 2)


# speedup vs baseline: 2.2366x; 2.2366x over previous
import functools

import jax
import jax.numpy as jnp
from jax import lax
from jax.experimental import pallas as pl
from jax.experimental.pallas import tpu as pltpu
from jax.experimental.pallas import tpu_sc as plsc

N = 16384
C_CAT = 26
C_NUM = 13
V = 100000
D = 16
NC = 2
NS = 16
NW = NC * NS

IDX_P = 4096
NUM_P = 2048


def _sc_body(embT_u, embT_i, idxT, numT, lwb, out2,
             acc_v, ib0, ib1, lw_v, sem, vsem):
    wid = lax.axis_index("s") * NC + lax.axis_index("c")
    t = wid // D
    d = wid % D

    pltpu.sync_copy(lwb.at[wid], lw_v)
    wk = [lw_v[pl.ds(k * D, D)] for k in range(C_NUM + 1)]

    def init_scope(num_v):
        nb = [num_v.at[0], num_v.at[1]]
        cps = [pltpu.async_copy(numT.at[pl.ds(t * D, D), pl.ds(p * NUM_P, NUM_P)],
                                nb[p % 2], vsem)
               for p in range(2)]
        for p in range(N // NUM_P):
            cps[p % 2].wait()
            buf = nb[p % 2]

            def nbody(j, _):
                val = buf[0, pl.ds(j * D, D)] * wk[0]
                for k in range(1, C_NUM + 1):
                    val = val + buf[k, pl.ds(j * D, D)] * wk[k]
                acc_v[pl.ds((p * NUM_P) + j * D, D)] = val
                return 0

            lax.fori_loop(0, NUM_P // D, nbody, 0, unroll=4)
            if p + 2 < N // NUM_P:
                cps[p % 2] = pltpu.async_copy(
                    numT.at[pl.ds(t * D, D), pl.ds((p + 2) * NUM_P, NUM_P)],
                    nb[p % 2], vsem)

    pl.run_scoped(init_scope, pltpu.VMEM((2, D, NUM_P), jnp.float32))

    NPIECE = N // IDX_P
    NG = C_CAT * NPIECE
    ib = [ib0, ib1]

    isems = [sem, vsem]

    def istart(g):
        c, p = divmod(g, NPIECE)
        return pltpu.async_copy(
            idxT.at[t, c, pl.ds(p * IDX_P, IDX_P)], ib[g % 2], isems[g % 2])

    def main_scope(vec_v):
        istart(0)
        for c in range(C_CAT):
            row = c * D + d

            @pl.when(t == 0)
            def _():
                pltpu.sync_copy(embT_u.at[row], vec_v)

            @pl.when(t > 0)
            def _():
                pltpu.sync_copy(embT_i.at[row], vec_v)

            for h in range(NPIECE):
                g = c * NPIECE + h
                pltpu.make_async_copy(
                    idxT.at[t, 0, pl.ds(0, IDX_P)], ib[g % 2],
                    isems[g % 2]).wait()
                if g + 1 < NG:
                    istart(g + 1)
                idx_v = ib[g % 2]
                base = h * IDX_P

                def gbody(j, _):
                    idxv = idx_v[pl.ds(j * D, D)]
                    g2 = plsc.load_gather(vec_v, [idxv])
                    a = base + j * D
                    acc_v[pl.ds(a, D)] = acc_v[pl.ds(a, D)] + g2
                    return 0

                lax.fori_loop(0, IDX_P // D, gbody, 0, unroll=3)

    pl.run_scoped(main_scope, pltpu.VMEM((V,), jnp.float32))

    pltpu.sync_copy(acc_v, out2.at[wid])


@jax.jit
def _run(embT_u, embT_i, idxT, numT, lwb):
    mesh = plsc.VectorSubcoreMesh(core_axis_name="c", subcore_axis_name="s")
    return pl.kernel(
        _sc_body,
        out_type=jax.ShapeDtypeStruct((NW, N), jnp.float32),
        mesh=mesh,
        scratch_types=[
            pltpu.VMEM((N,), jnp.float32),
            pltpu.VMEM((IDX_P,), jnp.int32),
            pltpu.VMEM((IDX_P,), jnp.int32),
            pltpu.VMEM((D * D,), jnp.float32),
            pltpu.SemaphoreType.DMA,
            pltpu.SemaphoreType.DMA,
        ],
        compiler_params=pltpu.CompilerParams(needs_layout_passes=False),
    )(embT_u, embT_i, idxT, numT, lwb)


def kernel(cat_user, num_user, cat_item, num_item,
           emb_user, lin_w_user, lin_b_user,
           emb_item, lin_w_item, lin_b_item):
    embT_u = emb_user.swapaxes(1, 2).reshape(C_CAT * D, V)
    embT_i = emb_item.swapaxes(1, 2).reshape(C_CAT * D, V)
    idxT = jnp.stack([cat_user.astype(jnp.int32).T,
                      cat_item.astype(jnp.int32).T])
    ones = jnp.ones((1, N), jnp.float32)
    zer = jnp.zeros((D - C_NUM - 1, N), jnp.float32)
    numT = jnp.concatenate(
        [num_user.T, ones, zer, num_item.T, ones, zer], axis=0)
    zw = jnp.zeros((D - C_NUM - 1, D), jnp.float32)
    lw_u = jnp.concatenate([lin_w_user, lin_b_user.sum(0)[None], zw], axis=0)
    lw_i = jnp.concatenate([lin_w_item, lin_b_item.sum(0)[None], zw], axis=0)
    lw2 = jnp.stack([lw_u, lw_i])
    lwb = jnp.repeat(lw2.transpose(0, 2, 1)[:, :, :, None], D, axis=3)
    lwb = lwb.reshape(NW, D * D)
    out2 = _run(embT_u, embT_i, idxT, numT, lwb)
    return out2.reshape(2, D, N).swapaxes(1, 2)


# speedup vs baseline: 2.2475x; 1.0049x over previous
import functools

import jax
import jax.numpy as jnp
from jax import lax
from jax.experimental import pallas as pl
from jax.experimental.pallas import tpu as pltpu
from jax.experimental.pallas import tpu_sc as plsc

N = 16384
C_CAT = 26
C_NUM = 13
V = 100000
D = 16
NC = 2
NS = 16
NW = NC * NS

IDX_P = 4096
NUM_P = 2048


def _sc_body(embT_u, embT_i, idxT, numT, lwb, out2,
             acc_v, ib0, ib1, lw_v, sem, vsem):
    wid = lax.axis_index("s") * NC + lax.axis_index("c")
    t = wid // D
    d = wid % D

    pltpu.sync_copy(lwb.at[wid], lw_v)
    wk = [lw_v[pl.ds(k * D, D)] for k in range(C_NUM + 1)]

    def init_scope(num_v):
        nb = [num_v.at[0], num_v.at[1]]
        cps = [pltpu.async_copy(numT.at[pl.ds(t * D, D), pl.ds(p * NUM_P, NUM_P)],
                                nb[p % 2], vsem)
               for p in range(2)]
        for p in range(N // NUM_P):
            cps[p % 2].wait()
            buf = nb[p % 2]

            def nbody(j, _):
                val = buf[0, pl.ds(j * D, D)] * wk[0]
                for k in range(1, C_NUM + 1):
                    val = val + buf[k, pl.ds(j * D, D)] * wk[k]
                acc_v[pl.ds((p * NUM_P) + j * D, D)] = val
                return 0

            lax.fori_loop(0, NUM_P // D, nbody, 0, unroll=4)
            if p + 2 < N // NUM_P:
                cps[p % 2] = pltpu.async_copy(
                    numT.at[pl.ds(t * D, D), pl.ds((p + 2) * NUM_P, NUM_P)],
                    nb[p % 2], vsem)

    pl.run_scoped(init_scope, pltpu.VMEM((2, D, NUM_P), jnp.float32))

    NPIECE = N // IDX_P
    NG = C_CAT * NPIECE
    ib = [ib0, ib1]

    isems = [sem, vsem]

    def istart(g):
        c, p = divmod(g, NPIECE)
        return pltpu.async_copy(
            idxT.at[t, c, pl.ds(p * IDX_P, IDX_P)], ib[g % 2], isems[g % 2])

    def main_scope(vec_v):
        istart(0)
        for c in range(C_CAT):
            row = c * D + d

            @pl.when(t == 0)
            def _():
                pltpu.sync_copy(embT_u.at[row], vec_v)

            @pl.when(t > 0)
            def _():
                pltpu.sync_copy(embT_i.at[row], vec_v)

            for h in range(NPIECE):
                g = c * NPIECE + h
                pltpu.make_async_copy(
                    idxT.at[t, 0, pl.ds(0, IDX_P)], ib[g % 2],
                    isems[g % 2]).wait()
                if g + 1 < NG:
                    istart(g + 1)
                idx_v = ib[g % 2]
                base = h * IDX_P

                def gbody(j, _):
                    idxv = idx_v[pl.ds(j * D, D)]
                    g2 = plsc.load_gather(vec_v, [idxv])
                    a = base + j * D
                    acc_v[pl.ds(a, D)] = acc_v[pl.ds(a, D)] + g2
                    return 0

                lax.fori_loop(0, IDX_P // D, gbody, 0, unroll=4)

    pl.run_scoped(main_scope, pltpu.VMEM((V,), jnp.float32))

    pltpu.sync_copy(acc_v, out2.at[wid])


@jax.jit
def _run(embT_u, embT_i, idxT, numT, lwb):
    mesh = plsc.VectorSubcoreMesh(core_axis_name="c", subcore_axis_name="s")
    return pl.kernel(
        _sc_body,
        out_type=jax.ShapeDtypeStruct((NW, N), jnp.float32),
        mesh=mesh,
        scratch_types=[
            pltpu.VMEM((N,), jnp.float32),
            pltpu.VMEM((IDX_P,), jnp.int32),
            pltpu.VMEM((IDX_P,), jnp.int32),
            pltpu.VMEM((D * D,), jnp.float32),
            pltpu.SemaphoreType.DMA,
            pltpu.SemaphoreType.DMA,
        ],
        compiler_params=pltpu.CompilerParams(needs_layout_passes=False),
    )(embT_u, embT_i, idxT, numT, lwb)


def kernel(cat_user, num_user, cat_item, num_item,
           emb_user, lin_w_user, lin_b_user,
           emb_item, lin_w_item, lin_b_item):
    embT_u = emb_user.swapaxes(1, 2).reshape(C_CAT * D, V)
    embT_i = emb_item.swapaxes(1, 2).reshape(C_CAT * D, V)
    idxT = jnp.stack([cat_user.astype(jnp.int32).T,
                      cat_item.astype(jnp.int32).T])
    ones = jnp.ones((1, N), jnp.float32)
    zer = jnp.zeros((D - C_NUM - 1, N), jnp.float32)
    numT = jnp.concatenate(
        [num_user.T, ones, zer, num_item.T, ones, zer], axis=0)
    zw = jnp.zeros((D - C_NUM - 1, D), jnp.float32)
    lw_u = jnp.concatenate([lin_w_user, lin_b_user.sum(0)[None], zw], axis=0)
    lw_i = jnp.concatenate([lin_w_item, lin_b_item.sum(0)[None], zw], axis=0)
    lw2 = jnp.stack([lw_u, lw_i])
    lwb = jnp.repeat(lw2.transpose(0, 2, 1)[:, :, :, None], D, axis=3)
    lwb = lwb.reshape(NW, D * D)
    out2 = _run(embT_u, embT_i, idxT, numT, lwb)
    return out2.reshape(2, D, N).swapaxes(1, 2)


# speedup vs baseline: 2.3060x; 1.0260x over previous
import functools

import jax
import jax.numpy as jnp
from jax import lax
from jax.experimental import pallas as pl
from jax.experimental.pallas import tpu as pltpu
from jax.experimental.pallas import tpu_sc as plsc

N = 16384
C_CAT = 26
C_NUM = 13
V = 100000
D = 16
NC = 2
NS = 16
NW = NC * NS

IDX_P = 4096
TBLK = 2048


def _lin_body(num_ref, w_ref, out_ref):
    out_ref[...] = lax.dot_general(
        w_ref[0], num_ref[...],
        (((0,), (0,)), ((), ())),
        precision=lax.Precision.HIGHEST,
        preferred_element_type=jnp.float32)


def _lin(numT, lw2):
    return pl.pallas_call(
        _lin_body,
        out_shape=jax.ShapeDtypeStruct((NW, N), jnp.float32),
        grid=(2, N // TBLK),
        in_specs=[
            pl.BlockSpec((D, TBLK), lambda t, j: (t, j)),
            pl.BlockSpec((1, D, D), lambda t, j: (t, 0, 0)),
        ],
        out_specs=pl.BlockSpec((D, TBLK), lambda t, j: (t, j)),
    )(numT, lw2)


def _sc_body(embT_u, embT_i, idxT, linT, out2,
             acc_v, ib0, ib1, sem, vsem, asem):
    wid = lax.axis_index("s") * NC + lax.axis_index("c")
    t = wid // D
    d = wid % D

    acp = pltpu.async_copy(linT.at[wid], acc_v, asem)

    NPIECE = N // IDX_P
    NG = C_CAT * NPIECE
    ib = [ib0, ib1]
    isems = [sem, vsem]

    def istart(g):
        c, p = divmod(g, NPIECE)
        return pltpu.async_copy(
            idxT.at[t, c, pl.ds(p * IDX_P, IDX_P)], ib[g % 2], isems[g % 2])

    def main_scope(vec_v):
        istart(0)
        for c in range(C_CAT):
            row = c * D + d

            @pl.when(t == 0)
            def _():
                pltpu.sync_copy(embT_u.at[row], vec_v)

            @pl.when(t > 0)
            def _():
                pltpu.sync_copy(embT_i.at[row], vec_v)

            if c == 0:
                acp.wait()

            for h in range(NPIECE):
                g = c * NPIECE + h
                pltpu.make_async_copy(
                    idxT.at[t, 0, pl.ds(0, IDX_P)], ib[g % 2],
                    isems[g % 2]).wait()
                if g + 1 < NG:
                    istart(g + 1)
                idx_v = ib[g % 2]
                base = h * IDX_P

                def gbody(j, _):
                    idxv = idx_v[pl.ds(j * D, D)]
                    g2 = plsc.load_gather(vec_v, [idxv])
                    a = base + j * D
                    acc_v[pl.ds(a, D)] = acc_v[pl.ds(a, D)] + g2
                    return 0

                lax.fori_loop(0, IDX_P // D, gbody, 0, unroll=4)

    pl.run_scoped(main_scope, pltpu.VMEM((V,), jnp.float32))

    pltpu.sync_copy(acc_v, out2.at[wid])


@jax.jit
def _run(embT_u, embT_i, idxT, linT):
    mesh = plsc.VectorSubcoreMesh(core_axis_name="c", subcore_axis_name="s")
    return pl.kernel(
        _sc_body,
        out_type=jax.ShapeDtypeStruct((NW, N), jnp.float32),
        mesh=mesh,
        scratch_types=[
            pltpu.VMEM((N,), jnp.float32),
            pltpu.VMEM((IDX_P,), jnp.int32),
            pltpu.VMEM((IDX_P,), jnp.int32),
            pltpu.SemaphoreType.DMA,
            pltpu.SemaphoreType.DMA,
            pltpu.SemaphoreType.DMA,
        ],
        compiler_params=pltpu.CompilerParams(needs_layout_passes=False),
    )(embT_u, embT_i, idxT, linT)


def kernel(cat_user, num_user, cat_item, num_item,
           emb_user, lin_w_user, lin_b_user,
           emb_item, lin_w_item, lin_b_item):
    embT_u = emb_user.swapaxes(1, 2).reshape(C_CAT * D, V)
    embT_i = emb_item.swapaxes(1, 2).reshape(C_CAT * D, V)
    idxT = jnp.stack([cat_user.astype(jnp.int32).T,
                      cat_item.astype(jnp.int32).T])
    ones = jnp.ones((1, N), jnp.float32)
    zer = jnp.zeros((D - C_NUM - 1, N), jnp.float32)
    numT = jnp.concatenate(
        [num_user.T, ones, zer, num_item.T, ones, zer], axis=0)
    zw = jnp.zeros((D - C_NUM - 1, D), jnp.float32)
    lw_u = jnp.concatenate([lin_w_user, lin_b_user.sum(0)[None], zw], axis=0)
    lw_i = jnp.concatenate([lin_w_item, lin_b_item.sum(0)[None], zw], axis=0)
    lw2 = jnp.stack([lw_u, lw_i])
    linT = _lin(numT, lw2)
    out2 = _run(embT_u, embT_i, idxT, linT)
    return out2.reshape(2, D, N).swapaxes(1, 2)
